# Initial kernel scaffold; baseline (speedup 1.0000x reference)
import jax
import jax.numpy as jnp
from jax.experimental import pallas as pl


def kernel(z, edge_index, W1, b1, W2, b2):
    raise NotImplementedError("write your pallas kernel here")


# speedup vs baseline: 8.7812x; 8.7812x over previous
import functools

import jax
import jax.numpy as jnp
from jax import lax
from jax.experimental import pallas as pl
from jax.experimental.pallas import tpu as pltpu
from jax.experimental.pallas import tpu_sc as plsc

_L = 16
_NC = 2
_NS = 16
_NW = _NC * _NS
_C = 80


def _precompute_tables(z, W1, b1):
    n, d = z.shape
    h = W1.shape[0]

    def body(z_ref, w1_ref, b1_ref, za_ref, zb_ref):
        zz = z_ref[...]
        w1 = w1_ref[...]
        za = lax.dot_general(zz, w1[:, :d], (((1,), (1,)), ((), ())),
                             preferred_element_type=jnp.float32)
        zb = lax.dot_general(zz, w1[:, d:], (((1,), (1,)), ((), ())),
                             preferred_element_type=jnp.float32)
        za_ref[...] = za + b1_ref[...]
        zb_ref[...] = zb

    return pl.pallas_call(
        body,
        out_shape=(jax.ShapeDtypeStruct((n, h), jnp.float32),
                   jax.ShapeDtypeStruct((n, h), jnp.float32)),
    )(z, W1, b1.reshape(1, h))


@functools.lru_cache(maxsize=None)
def _make_sc_scorer(n_edges, hid):
    epw = n_edges // _NW
    nch = epw // _C
    nk = hid // _L

    mesh = plsc.VectorSubcoreMesh(core_axis_name="c", subcore_axis_name="s")

    @functools.partial(
        pl.kernel,
        out_type=jax.ShapeDtypeStruct((_NW, nch, _C), jnp.float32),
        mesh=mesh,
        compiler_params=pltpu.CompilerParams(
            needs_layout_passes=False, use_tc_tiling_on_sc=False),
        scratch_types=[
            pltpu.VMEM((nch, _C), jnp.int32),
            pltpu.VMEM((nch, _C), jnp.int32),
            pltpu.VMEM((2, _C, hid), jnp.float32),
            pltpu.VMEM((2, _C, hid), jnp.float32),
            pltpu.VMEM((nch, _C), jnp.float32),
            pltpu.VMEM((_C,), jnp.float32),
            pltpu.SemaphoreType.DMA,
            pltpu.SemaphoreType.DMA,
            pltpu.SemaphoreType.DMA,
            pltpu.SemaphoreType.DMA,
        ],
    )
    def scorer(eidx_hbm, za_hbm, zb_hbm, wv_hbm, out_hbm,
               src_v, dst_v, rows_a, rows_b, out_v, wv_v,
               sa0, sa1, sb0, sb1):
        wid = lax.axis_index("s") * _NC + lax.axis_index("c")
        pltpu.sync_copy(eidx_hbm.at[0, wid], src_v)
        pltpu.sync_copy(eidx_hbm.at[1, wid], dst_v)
        pltpu.sync_copy(wv_hbm, wv_v)

        sems_a = (sa0, sa1)
        sems_b = (sb0, sb1)

        def gather_start(g, slot):
            pltpu.async_copy(za_hbm.at[src_v.at[g]], rows_a.at[slot], sems_a[slot])
            pltpu.async_copy(zb_hbm.at[dst_v.at[g]], rows_b.at[slot], sems_b[slot])

        def gather_wait(g, slot):
            pltpu.make_async_copy(
                za_hbm.at[src_v.at[g]], rows_a.at[slot], sems_a[slot]).wait()
            pltpu.make_async_copy(
                zb_hbm.at[dst_v.at[g]], rows_b.at[slot], sems_b[slot]).wait()

        w = [wv_v[pl.ds(k * _L, _L)] for k in range(nk)]
        b2v = wv_v[pl.ds(hid, _L)]
        zero = jnp.zeros((_L,), jnp.float32)
        one = jnp.ones((_L,), jnp.float32)
        lane = lax.iota(jnp.int32, _L)
        masks = [lane == e for e in range(_L)]

        def compute(g, slot):
            for blk in range(_C // _L):
                acc = zero
                for e in range(_L):
                    r = blk * _L + e
                    p = zero
                    for k in range(nk):
                        a = rows_a[slot, r, pl.ds(k * _L, _L)]
                        b = rows_b[slot, r, pl.ds(k * _L, _L)]
                        p = p + jnp.maximum(a + b, 0.0) * w[k]
                    acc = jnp.where(masks[e], jnp.sum(p), acc)
                x = acc + b2v
                out_v[g, pl.ds(blk * _L, _L)] = one / (one + jnp.exp(-x))

        gather_start(0, 0)

        def ring_body(i, carry):
            g0 = 2 * i
            g1 = g0 + 1
            gather_start(g1, 1)
            gather_wait(g0, 0)
            compute(g0, 0)
            gather_start(g1 + 1, 0)
            gather_wait(g1, 1)
            compute(g1, 1)
            return carry

        lax.fori_loop(0, (nch - 1) // 2, ring_body, 0)
        gather_wait(nch - 1, 0)
        compute(nch - 1, 0)

        pltpu.sync_copy(out_v, out_hbm.at[wid])

    return scorer


def kernel(z, edge_index, W1, b1, W2, b2):
    n_edges = edge_index.shape[1]
    hid = W1.shape[0]
    za, zb = _precompute_tables(z, W1, b1)
    eidx = edge_index.astype(jnp.int32).reshape(2, _NW, n_edges // (_NW * _C), _C)
    wv = jnp.concatenate(
        [W2.reshape(-1), jnp.broadcast_to(b2, (_L,))]).astype(jnp.float32)
    out = _make_sc_scorer(n_edges, hid)(eidx, za, zb, wv)
    return out.reshape(-1)


# speedup vs baseline: 8.8845x; 1.0118x over previous
import functools

import jax
import jax.numpy as jnp
from jax import lax
from jax.experimental import pallas as pl
from jax.experimental.pallas import tpu as pltpu
from jax.experimental.pallas import tpu_sc as plsc

_L = 16
_NC = 2
_NS = 16
_NW = _NC * _NS
_C = 80
_D = 5


def _precompute_tables(z, W1, b1):
    n, d = z.shape
    h = W1.shape[0]

    def body(z_ref, w1_ref, b1_ref, za_ref, zb_ref):
        zz = z_ref[...]
        w1 = w1_ref[...]
        za = lax.dot_general(zz, w1[:, :d], (((1,), (1,)), ((), ())),
                             preferred_element_type=jnp.float32)
        zb = lax.dot_general(zz, w1[:, d:], (((1,), (1,)), ((), ())),
                             preferred_element_type=jnp.float32)
        za_ref[...] = (za + b1_ref[...]).astype(jnp.bfloat16)
        zb_ref[...] = zb.astype(jnp.bfloat16)

    return pl.pallas_call(
        body,
        out_shape=(jax.ShapeDtypeStruct((n, h), jnp.bfloat16),
                   jax.ShapeDtypeStruct((n, h), jnp.bfloat16)),
    )(z, W1, b1.reshape(1, h))


@functools.lru_cache(maxsize=None)
def _make_sc_scorer(n_edges, hid):
    epw = n_edges // _NW
    nch = epw // _C
    nk = hid // _L

    mesh = plsc.VectorSubcoreMesh(core_axis_name="c", subcore_axis_name="s")

    @functools.partial(
        pl.kernel,
        out_type=jax.ShapeDtypeStruct((_NW, nch, _C), jnp.float32),
        mesh=mesh,
        compiler_params=pltpu.CompilerParams(
            needs_layout_passes=False, use_tc_tiling_on_sc=False),
        scratch_types=[
            pltpu.VMEM((nch, _C), jnp.int32),
            pltpu.VMEM((nch, _C), jnp.int32),
            pltpu.VMEM((_D, _C, hid), jnp.bfloat16),
            pltpu.VMEM((_D, _C, hid), jnp.bfloat16),
            pltpu.VMEM((nch, _C), jnp.float32),
            pltpu.VMEM((_C,), jnp.float32),
        ] + [pltpu.SemaphoreType.DMA] * (2 * _D),
    )
    def scorer(eidx_hbm, za_hbm, zb_hbm, wv_hbm, out_hbm,
               src_v, dst_v, rows_a, rows_b, out_v, wv_v,
               *sems):
        wid = lax.axis_index("s") * _NC + lax.axis_index("c")
        pltpu.sync_copy(eidx_hbm.at[0, wid], src_v)
        pltpu.sync_copy(eidx_hbm.at[1, wid], dst_v)
        pltpu.sync_copy(wv_hbm, wv_v)

        sems_a = sems[:_D]
        sems_b = sems[_D:]

        def gather_start(g, slot):
            pltpu.async_copy(za_hbm.at[src_v.at[g]], rows_a.at[slot], sems_a[slot])
            pltpu.async_copy(zb_hbm.at[dst_v.at[g]], rows_b.at[slot], sems_b[slot])

        def gather_wait(g, slot):
            pltpu.make_async_copy(
                za_hbm.at[src_v.at[g]], rows_a.at[slot], sems_a[slot]).wait()
            pltpu.make_async_copy(
                zb_hbm.at[dst_v.at[g]], rows_b.at[slot], sems_b[slot]).wait()

        w = [wv_v[pl.ds(k * _L, _L)] for k in range(nk)]
        b2v = wv_v[pl.ds(hid, _L)]
        zero = jnp.zeros((_L,), jnp.float32)
        one = jnp.ones((_L,), jnp.float32)
        zero_b = jnp.zeros((2 * _L,), jnp.bfloat16)
        lane = lax.iota(jnp.int32, _L)
        masks = [lane == e for e in range(_L)]

        def compute(g, slot):
            for blk in range(_C // _L):
                acc = zero
                for e in range(_L):
                    r = blk * _L + e
                    p = zero
                    for h in range(nk // 2):
                        a = rows_a[slot, r, pl.ds(h * 2 * _L, 2 * _L)]
                        b = rows_b[slot, r, pl.ds(h * 2 * _L, 2 * _L)]
                        t = jnp.maximum(a + b, zero_b)
                        te, to = plsc.unpack(
                            t, format=plsc.PackFormat.INTERLEAVED,
                            preferred_element_type=jnp.float32)
                        p = p + te * w[2 * h] + to * w[2 * h + 1]
                    acc = jnp.where(masks[e], jnp.sum(p), acc)
                x = acc + b2v
                out_v[g, pl.ds(blk * _L, _L)] = one / (one + jnp.exp(-x))

        for s in range(_D - 1):
            gather_start(s, s)

        def ring_body(i, carry):
            for j in range(_D):
                g = _D * i + j
                gather_start(g + _D - 1, (j + _D - 1) % _D)
                gather_wait(g, j)
                compute(g, j)
            return carry

        lax.fori_loop(0, nch // _D - 1, ring_body, 0)
        base = nch - _D
        for j in range(_D):
            g = base + j
            if j < 1:
                gather_start(g + _D - 1, (j + _D - 1) % _D)
            gather_wait(g, j)
            compute(g, j)

        pltpu.sync_copy(out_v, out_hbm.at[wid])

    return scorer


def kernel(z, edge_index, W1, b1, W2, b2):
    n_edges = edge_index.shape[1]
    hid = W1.shape[0]
    za, zb = _precompute_tables(z, W1, b1)
    eidx = edge_index.astype(jnp.int32).reshape(2, _NW, n_edges // (_NW * _C), _C)
    w2f = W2.reshape(-1).astype(jnp.float32)
    wp = jnp.concatenate(
        [w2f[g * 2 * _L + p0:(g + 1) * 2 * _L:2] for g in range(hid // (2 * _L))
         for p0 in (0, 1)])
    wv = jnp.concatenate([wp, jnp.broadcast_to(b2, (_L,))]).astype(jnp.float32)
    out = _make_sc_scorer(n_edges, hid)(eidx, za, zb, wv)
    return out.reshape(-1)
